# baseline (device time: 49327 ns/iter reference)
import jax
import jax.numpy as jnp
from jax import lax
from jax.experimental import pallas as pl
from jax.experimental.pallas import tpu as pltpu

N_DEV = 4
DH = 64


def _ring_allreduce(partial):
    M, N = partial.shape
    chunk = M // N_DEV

    def body(p_ref, out_ref, comm_ref, send_sems, recv_sems):
        i = lax.axis_index("i")
        right = lax.rem(i + 1, N_DEV)
        left = lax.rem(i + N_DEV - 1, N_DEV)

        barrier = pltpu.get_barrier_semaphore()
        for nbr in (left, right):
            pl.semaphore_signal(
                barrier, inc=1,
                device_id=(nbr,), device_id_type=pl.DeviceIdType.MESH,
            )
        pl.semaphore_wait(barrier, 2)

        out_ref[...] = p_ref[...]

        for s in range(N_DEV - 1):
            send_c = lax.rem(i - s + 2 * N_DEV, N_DEV)
            recv_c = lax.rem(i - s - 1 + 2 * N_DEV, N_DEV)
            rdma = pltpu.make_async_remote_copy(
                src_ref=out_ref.at[pl.ds(send_c * chunk, chunk), :],
                dst_ref=comm_ref.at[s],
                send_sem=send_sems.at[s],
                recv_sem=recv_sems.at[s],
                device_id=(right,),
                device_id_type=pl.DeviceIdType.MESH,
            )
            rdma.start()
            rdma.wait()
            rows = pl.ds(recv_c * chunk, chunk)
            out_ref[rows, :] = out_ref[rows, :] + comm_ref[s]

        for s in range(N_DEV - 1):
            send_c = lax.rem(i + 1 - s + 2 * N_DEV, N_DEV)
            rows = pl.ds(send_c * chunk, chunk)
            rdma = pltpu.make_async_remote_copy(
                src_ref=out_ref.at[rows, :],
                dst_ref=out_ref.at[rows, :],
                send_sem=send_sems.at[N_DEV - 1 + s],
                recv_sem=recv_sems.at[N_DEV - 1 + s],
                device_id=(right,),
                device_id_type=pl.DeviceIdType.MESH,
            )
            rdma.start()
            rdma.wait()

    return pl.pallas_call(
        body,
        out_shape=jax.ShapeDtypeStruct((M, N), jnp.float32),
        in_specs=[pl.BlockSpec(memory_space=pltpu.VMEM)],
        out_specs=pl.BlockSpec(memory_space=pltpu.VMEM),
        scratch_shapes=[
            pltpu.VMEM((N_DEV - 1, chunk, N), jnp.float32),
            pltpu.SemaphoreType.DMA((2 * (N_DEV - 1),)),
            pltpu.SemaphoreType.DMA((2 * (N_DEV - 1),)),
        ],
        compiler_params=pltpu.CompilerParams(collective_id=0),
    )(partial)


def kernel(x, Wq, Wo, Wk, Wv):
    B, Sq, D = x.shape
    Hq_local = Wq.shape[1] // DH

    i = lax.axis_index("i")
    bf = jnp.bfloat16
    xb = x.astype(bf)

    Q = (xb @ Wq.astype(bf)).reshape(B, Sq, Hq_local, DH)
    kv0 = 2 * i
    Wk_l = lax.dynamic_slice_in_dim(Wk.astype(bf), kv0 * DH, 2 * DH, axis=1)
    Wv_l = lax.dynamic_slice_in_dim(Wv.astype(bf), kv0 * DH, 2 * DH, axis=1)
    K = (xb @ Wk_l).reshape(B, Sq, 2, DH)
    V = (xb @ Wv_l).reshape(B, Sq, 2, DH)
    K = jnp.repeat(K, 4, axis=2)
    V = jnp.repeat(V, 4, axis=2)

    s = jnp.einsum(
        "bihd,bjhd->bhij", Q, K, preferred_element_type=jnp.float32
    ) * 0.125
    p = jax.nn.softmax(s, axis=-1)
    o = jnp.einsum("bhij,bjhd->bihd", p.astype(bf), V)
    o = o.reshape(B, Sq, Hq_local * DH)

    partial = jnp.einsum(
        "bsf,fd->bsd", o, Wo.astype(bf), preferred_element_type=jnp.float32
    )

    out = _ring_allreduce(partial.reshape(B * Sq, D))
    return out.reshape(B, Sq, D)


# device time: 33335 ns/iter; 1.4797x vs baseline; 1.4797x over previous
import jax
import jax.numpy as jnp
from jax import lax
from jax.experimental import pallas as pl
from jax.experimental.pallas import tpu as pltpu

N_DEV = 4
DH = 64


def _butterfly_allreduce(partial):
    M, N = partial.shape
    half = M // 2
    quart = M // 4

    def body(p_ref, out_ref, buf_ref, comm_a, comm_b, send_sems, recv_sems):
        i = lax.axis_index("i")
        partner_a = i ^ 1
        partner_b = 3 - i
        keep_a = (i + 1) // 2 % 2
        keep_b = i // 2

        barrier = pltpu.get_barrier_semaphore()
        for nbr in (partner_a, partner_b):
            pl.semaphore_signal(
                barrier, inc=1,
                device_id=(nbr,), device_id_type=pl.DeviceIdType.MESH,
            )
        pl.semaphore_wait(barrier, 2)

        buf_ref[...] = p_ref[...]

        keep_rows = pl.ds(keep_a * half, half)
        rdma = pltpu.make_async_remote_copy(
            src_ref=buf_ref.at[pl.ds((1 - keep_a) * half, half), :],
            dst_ref=comm_a,
            send_sem=send_sems.at[0],
            recv_sem=recv_sems.at[0],
            device_id=(partner_a,),
            device_id_type=pl.DeviceIdType.MESH,
        )
        rdma.start()
        rdma.wait()
        buf_ref[keep_rows, :] = buf_ref[keep_rows, :] + comm_a[...]

        q_own = 2 * keep_a + keep_b
        q_send = 2 * keep_a + (1 - keep_b)
        own_rows = pl.ds(q_own * quart, quart)
        rdma = pltpu.make_async_remote_copy(
            src_ref=buf_ref.at[pl.ds(q_send * quart, quart), :],
            dst_ref=comm_b,
            send_sem=send_sems.at[1],
            recv_sem=recv_sems.at[1],
            device_id=(partner_b,),
            device_id_type=pl.DeviceIdType.MESH,
        )
        rdma.start()
        rdma.wait()
        buf_ref[own_rows, :] = buf_ref[own_rows, :] + comm_b[...]

        rdma = pltpu.make_async_remote_copy(
            src_ref=buf_ref.at[own_rows, :],
            dst_ref=buf_ref.at[own_rows, :],
            send_sem=send_sems.at[2],
            recv_sem=recv_sems.at[2],
            device_id=(partner_b,),
            device_id_type=pl.DeviceIdType.MESH,
        )
        rdma.start()
        rdma.wait()

        rdma = pltpu.make_async_remote_copy(
            src_ref=buf_ref.at[keep_rows, :],
            dst_ref=buf_ref.at[keep_rows, :],
            send_sem=send_sems.at[3],
            recv_sem=recv_sems.at[3],
            device_id=(partner_a,),
            device_id_type=pl.DeviceIdType.MESH,
        )
        rdma.start()
        rdma.wait()

        out_ref[...] = buf_ref[...].astype(jnp.float32)

    return pl.pallas_call(
        body,
        out_shape=jax.ShapeDtypeStruct((M, N), jnp.float32),
        in_specs=[pl.BlockSpec(memory_space=pltpu.VMEM)],
        out_specs=pl.BlockSpec(memory_space=pltpu.VMEM),
        scratch_shapes=[
            pltpu.VMEM((M, N), jnp.bfloat16),
            pltpu.VMEM((half, N), jnp.bfloat16),
            pltpu.VMEM((quart, N), jnp.bfloat16),
            pltpu.SemaphoreType.DMA((4,)),
            pltpu.SemaphoreType.DMA((4,)),
        ],
        compiler_params=pltpu.CompilerParams(collective_id=0),
    )(partial)


def kernel(x, Wq, Wo, Wk, Wv):
    B, Sq, D = x.shape
    Hq_local = Wq.shape[1] // DH

    i = lax.axis_index("i")
    bf = jnp.bfloat16
    xb = x.astype(bf)

    Q = (xb @ Wq.astype(bf)).reshape(B, Sq, Hq_local, DH)
    kv0 = 2 * i
    Wk_l = lax.dynamic_slice_in_dim(Wk.astype(bf), kv0 * DH, 2 * DH, axis=1)
    Wv_l = lax.dynamic_slice_in_dim(Wv.astype(bf), kv0 * DH, 2 * DH, axis=1)
    K = (xb @ Wk_l).reshape(B, Sq, 2, DH)
    V = (xb @ Wv_l).reshape(B, Sq, 2, DH)
    K = jnp.repeat(K, 4, axis=2)
    V = jnp.repeat(V, 4, axis=2)

    s = jnp.einsum(
        "bihd,bjhd->bhij", Q, K, preferred_element_type=jnp.float32
    ) * 0.125
    p = jax.nn.softmax(s, axis=-1)
    o = jnp.einsum("bhij,bjhd->bihd", p.astype(bf), V)
    o = o.reshape(B, Sq, Hq_local * DH)

    partial = (o @ Wo.astype(bf)).astype(bf)

    out = _butterfly_allreduce(partial.reshape(B * Sq, D))
    return out.reshape(B, Sq, D)


# device time: 27444 ns/iter; 1.7974x vs baseline; 1.2147x over previous
import jax
import jax.numpy as jnp
from jax import lax
from jax.experimental import pallas as pl
from jax.experimental.pallas import tpu as pltpu

N_DEV = 4
DH = 64


def _fused_matmul_allreduce(o, Wo):
    M, K = o.shape
    N = Wo.shape[1]
    HALF, QUART, CW = M // 2, M // 4, N // 2
    bf = jnp.bfloat16

    def body(o_ref, wo_ref, out_ref, buf_ref, c1a, c1b, c2a, c2b,
             send_sems, recv_sems):
        i = lax.axis_index("i")
        pa = i ^ 1
        pb = 3 - i
        k1a = (i + 1) // 2 % 2
        k2a = i // 2
        k1b = i // 2
        k2b = i % 2

        barrier = pltpu.get_barrier_semaphore()
        for nbr in (pa, pb):
            pl.semaphore_signal(
                barrier, inc=1,
                device_id=(nbr,), device_id_type=pl.DeviceIdType.MESH,
            )
        pl.semaphore_wait(barrier, 2)

        buf_ref[...] = jnp.dot(
            o_ref[...], wo_ref[...], preferred_element_type=jnp.float32
        ).astype(bf)

        def xchg(slot, partner, r0, nr, c0, dst_comm=None):
            src = buf_ref.at[pl.ds(r0, nr), pl.ds(c0, CW)]
            dst = src if dst_comm is None else dst_comm
            r = pltpu.make_async_remote_copy(
                src_ref=src, dst_ref=dst,
                send_sem=send_sems.at[slot], recv_sem=recv_sems.at[slot],
                device_id=(partner,), device_id_type=pl.DeviceIdType.MESH,
            )
            r.start()
            return r

        def acc(r0, nr, c0, comm):
            rows, cols = pl.ds(r0, nr), pl.ds(c0, CW)
            buf_ref[rows, cols] = buf_ref[rows, cols] + comm[...]

        ra = xchg(0, pa, (1 - k1a) * HALF, HALF, 0, c1a)
        rb = xchg(1, pb, (1 - k1b) * HALF, HALF, CW, c1b)
        ra.wait()
        acc(k1a * HALF, HALF, 0, c1a)
        rb.wait()
        acc(k1b * HALF, HALF, CW, c1b)

        ra = xchg(2, pb, (2 * k1a + 1 - k2a) * QUART, QUART, 0, c2a)
        rb = xchg(3, pa, (2 * k1b + 1 - k2b) * QUART, QUART, CW, c2b)
        ra.wait()
        acc((2 * k1a + k2a) * QUART, QUART, 0, c2a)
        rb.wait()
        acc((2 * k1b + k2b) * QUART, QUART, CW, c2b)

        ra = xchg(4, pb, (2 * k1a + k2a) * QUART, QUART, 0)
        rb = xchg(5, pa, (2 * k1b + k2b) * QUART, QUART, CW)
        ra.wait()
        rb.wait()

        ra = xchg(6, pa, k1a * HALF, HALF, 0)
        rb = xchg(7, pb, k1b * HALF, HALF, CW)
        ra.wait()
        rb.wait()

        out_ref[...] = buf_ref[...].astype(jnp.float32)

    return pl.pallas_call(
        body,
        out_shape=jax.ShapeDtypeStruct((M, N), jnp.float32),
        in_specs=[
            pl.BlockSpec(memory_space=pltpu.VMEM),
            pl.BlockSpec(memory_space=pltpu.VMEM),
        ],
        out_specs=pl.BlockSpec(memory_space=pltpu.VMEM),
        scratch_shapes=[
            pltpu.VMEM((M, N), bf),
            pltpu.VMEM((HALF, CW), bf),
            pltpu.VMEM((HALF, CW), bf),
            pltpu.VMEM((QUART, CW), bf),
            pltpu.VMEM((QUART, CW), bf),
            pltpu.SemaphoreType.DMA((8,)),
            pltpu.SemaphoreType.DMA((8,)),
        ],
        compiler_params=pltpu.CompilerParams(collective_id=0),
    )(o, Wo)


def kernel(x, Wq, Wo, Wk, Wv):
    B, Sq, D = x.shape
    Hq_local = Wq.shape[1] // DH

    i = lax.axis_index("i")
    bf = jnp.bfloat16
    xb = x.astype(bf)

    Q = (xb @ Wq.astype(bf)).reshape(B, Sq, 2, 4, DH)
    kv0 = 2 * i
    Wk_l = lax.dynamic_slice_in_dim(Wk, kv0 * DH, 2 * DH, axis=1).astype(bf)
    Wv_l = lax.dynamic_slice_in_dim(Wv, kv0 * DH, 2 * DH, axis=1).astype(bf)
    K = (xb @ Wk_l).reshape(B, Sq, 2, DH)
    V = (xb @ Wv_l).reshape(B, Sq, 2, DH)

    s = jnp.einsum(
        "bikgd,bjkd->bkgij", Q, K, preferred_element_type=jnp.float32
    ) * 0.125
    p = jax.nn.softmax(s, axis=-1)
    o = jnp.einsum("bkgij,bjkd->bikgd", p.astype(bf), V)
    o = o.reshape(B * Sq, Hq_local * DH)

    out = _fused_matmul_allreduce(o, Wo.astype(bf))
    return out.reshape(B, Sq, D)
